# Initial kernel scaffold; baseline (speedup 1.0000x reference)
import jax
import jax.numpy as jnp
from jax.experimental import pallas as pl


def kernel(hidden_states, token_type_ids, gate_v, up_v, down_v, gate_l, up_l, down_l):
    raise NotImplementedError("write your pallas kernel here")


# speedup vs baseline: 1.0548x; 1.0548x over previous
import jax
import jax.numpy as jnp
from jax.experimental import pallas as pl
from jax.experimental.pallas import tpu as pltpu


def _fused_mlp_kernel(t0_ref, t1_ref, x_ref, gv_ref, uv_ref, gl_ref,
                      ul_ref, dv_ref, dl_ref, out_ref):
    i = pl.program_id(0)
    maskf = ((t0_ref[:] == 1) & (t1_ref[:] == 1)).astype(jnp.float32)

    x = x_ref[:].astype(jnp.bfloat16)
    f32 = jnp.float32

    hv = jax.nn.silu(
        jnp.dot(x, gv_ref[:].astype(jnp.bfloat16), preferred_element_type=f32)
    ) * jnp.dot(x, uv_ref[:].astype(jnp.bfloat16), preferred_element_type=f32)
    hl = jax.nn.silu(
        jnp.dot(x, gl_ref[:].astype(jnp.bfloat16), preferred_element_type=f32)
    ) * jnp.dot(x, ul_ref[:].astype(jnp.bfloat16), preferred_element_type=f32)

    hv = (hv * maskf).astype(jnp.bfloat16)
    hl = (hl * (1.0 - maskf)).astype(jnp.bfloat16)
    contrib = (
        jnp.dot(hv, dv_ref[:].astype(jnp.bfloat16), preferred_element_type=f32)
        + jnp.dot(hl, dl_ref[:].astype(jnp.bfloat16), preferred_element_type=f32)
    )

    @pl.when(i == 0)
    def _():
        out_ref[:] = contrib

    @pl.when(i > 0)
    def _():
        out_ref[:] += contrib


def kernel(hidden_states, token_type_ids, gate_v, up_v, down_v,
           gate_l, up_l, down_l):
    B, L, D = hidden_states.shape
    I = gate_v.shape[1]
    N = B * L
    C = 256
    steps = I // C
    assert steps * C == I

    x = hidden_states.reshape(N, D)
    t0 = token_type_ids.reshape(N, 1)
    t_next = jnp.concatenate(
        [token_type_ids[:, 1:],
         jnp.full((B, 1), -1, dtype=token_type_ids.dtype)], axis=1)
    t1 = t_next.reshape(N, 1)

    out = pl.pallas_call(
        _fused_mlp_kernel,
        grid=(steps,),
        in_specs=[
            pl.BlockSpec((N, 1), lambda i: (0, 0)),
            pl.BlockSpec((N, 1), lambda i: (0, 0)),
            pl.BlockSpec((N, D), lambda i: (0, 0)),
            pl.BlockSpec((D, C), lambda i: (0, i)),
            pl.BlockSpec((D, C), lambda i: (0, i)),
            pl.BlockSpec((D, C), lambda i: (0, i)),
            pl.BlockSpec((D, C), lambda i: (0, i)),
            pl.BlockSpec((C, D), lambda i: (i, 0)),
            pl.BlockSpec((C, D), lambda i: (i, 0)),
        ],
        out_specs=pl.BlockSpec((N, D), lambda i: (0, 0)),
        out_shape=jax.ShapeDtypeStruct((N, D), jnp.float32),
        compiler_params=pltpu.CompilerParams(
            dimension_semantics=("arbitrary",),
        ),
    )(t0, t1, x, gate_v, up_v, gate_l, up_l, down_v, down_l)

    return out.reshape(B, L, D)
